# baseline (speedup 1.0000x reference)
import functools

import jax
import jax.numpy as jnp
from jax import lax
from jax.experimental import pallas as pl
from jax.experimental.pallas import tpu as pltpu
from jax.experimental.pallas import tpu_sc as plsc

N = 16384
LATENT = 16
NUM_METABOLITES = 128

NUM_CORES = 2
NUM_SUBCORES = 16
NW = NUM_CORES * NUM_SUBCORES
BPW = N // NW
NCH = BPW // 128

_SC_MESH = plsc.VectorSubcoreMesh(
    core_axis_name="c", subcore_axis_name="s",
    num_cores=NUM_CORES, num_subcores=NUM_SUBCORES,
)


def _sc_encode(x_hbm, emb_mu_hbm, emb_lv_hbm, bias_mu_hbm, bias_lv_hbm,
               eps_emb_hbm, eps_bias_hbm, code_hbm,
               idx_v, mu_v, lv_v, eps_v, out_v,
               bmu_v, blv_v, epsb_v, bias_v, sem):
    wid = lax.axis_index("s") * NUM_CORES + lax.axis_index("c")
    base = wid * BPW

    pltpu.sync_copy(x_hbm.at[pl.ds(wid * NCH, NCH)], idx_v)
    gathers = []
    for j in range(NCH):
        row = pl.ds(j * 128, 128)
        gathers.append(
            pltpu.async_copy(emb_mu_hbm.at[idx_v.at[j]], mu_v.at[row], sem))
        gathers.append(
            pltpu.async_copy(emb_lv_hbm.at[idx_v.at[j]], lv_v.at[row], sem))
        gathers.append(
            pltpu.async_copy(bias_mu_hbm.at[idx_v.at[j]], bmu_v.at[row], sem))
        gathers.append(
            pltpu.async_copy(bias_lv_hbm.at[idx_v.at[j]], blv_v.at[row], sem))
    pltpu.sync_copy(eps_emb_hbm.at[pl.ds(base, BPW)], eps_v)
    pltpu.sync_copy(eps_bias_hbm.at[pl.ds(base, BPW)], epsb_v)
    for g in gathers:
        g.wait()

    def bias_body(j):
        sl = pl.ds(j * 16, 16)
        bias_v[sl] = bmu_v[sl] + jnp.exp(0.5 * blv_v[sl]) * epsb_v[sl]
    plsc.parallel_loop(0, BPW // 16, 1, unroll=4)(bias_body)

    def row_body(i):
        splat = plsc.load_gather(bias_v, [jnp.full((16,), 0, jnp.int32) + i])
        out_v[i, :] = mu_v[i, :] + jnp.exp(0.5 * lv_v[i, :]) * eps_v[i, :] + splat
    plsc.parallel_loop(0, BPW, 1, unroll=4)(row_body)

    pltpu.sync_copy(out_v, code_hbm.at[pl.ds(base, BPW)])


_sc_encode_call = functools.partial(
    pl.kernel,
    out_type=jax.ShapeDtypeStruct((N, LATENT), jnp.float32),
    mesh=_SC_MESH,
    scratch_types=[
        pltpu.VMEM((NCH, 128), jnp.int32),
        pltpu.VMEM((BPW, LATENT), jnp.float32),
        pltpu.VMEM((BPW, LATENT), jnp.float32),
        pltpu.VMEM((BPW, LATENT), jnp.float32),
        pltpu.VMEM((BPW, LATENT), jnp.float32),
        pltpu.VMEM((BPW,), jnp.float32),
        pltpu.VMEM((BPW,), jnp.float32),
        pltpu.VMEM((BPW,), jnp.float32),
        pltpu.VMEM((BPW,), jnp.float32),
        pltpu.SemaphoreType.DMA,
    ],
    compiler_params=pltpu.CompilerParams(
        needs_layout_passes=False, use_tc_tiling_on_sc=False),
)(_sc_encode)


def _tc_decode_body(code_ref, eps_ref, wmu_ref, wlv_ref, bmu_ref, blv_ref,
                    out_ref):
    code = code_ref[...]
    mean = jnp.dot(code, wmu_ref[...], preferred_element_type=jnp.float32)
    logvar = jnp.dot(code, wlv_ref[...], preferred_element_type=jnp.float32)
    mean = mean + bmu_ref[...]
    logvar = logvar + blv_ref[...]
    out_ref[...] = mean + jnp.exp(0.5 * logvar) * eps_ref[...]


def _tc_decode(code, eps_dec, W_mu, W_lv, b_mu, b_lv):
    blk = 2048
    grid = (N // blk,)
    return pl.pallas_call(
        _tc_decode_body,
        grid=grid,
        in_specs=[
            pl.BlockSpec((blk, LATENT), lambda i: (i, 0)),
            pl.BlockSpec((blk, NUM_METABOLITES), lambda i: (i, 0)),
            pl.BlockSpec((LATENT, NUM_METABOLITES), lambda i: (0, 0)),
            pl.BlockSpec((LATENT, NUM_METABOLITES), lambda i: (0, 0)),
            pl.BlockSpec((1, NUM_METABOLITES), lambda i: (0, 0)),
            pl.BlockSpec((1, NUM_METABOLITES), lambda i: (0, 0)),
        ],
        out_specs=pl.BlockSpec((blk, NUM_METABOLITES), lambda i: (i, 0)),
        out_shape=jax.ShapeDtypeStruct((N, NUM_METABOLITES), jnp.float32),
        compiler_params=pltpu.CompilerParams(
            dimension_semantics=("parallel",),
        ),
    )(code, eps_dec, W_mu, W_lv, b_mu, b_lv)


def kernel(x, emb_mu, emb_lv, bias_mu, bias_lv, W_mu, b_mu, W_lv, b_lv,
           eps_emb, eps_bias, eps_dec):
    code = _sc_encode_call(
        x.reshape(-1, 128), emb_mu, emb_lv,
        bias_mu.reshape(-1), bias_lv.reshape(-1),
        eps_emb, eps_bias.reshape(-1),
    )
    return _tc_decode(code, eps_dec, W_mu, W_lv,
                      b_mu.reshape(1, -1), b_lv.reshape(1, -1))
